# baseline (speedup 1.0000x reference)
import functools

import jax
import jax.numpy as jnp
from jax.experimental import pallas as pl
from jax.experimental.pallas import tpu as pltpu

_NBUF = 3


def _online_block(x_view, w, sc: int):
    bb, s, h = x_view.shape
    nc = s // sc
    m = d = acc = None
    for c in range(nc):
        xc = x_view[:, c * sc:(c + 1) * sc, :]
        scores = jnp.sum(jnp.tanh(xc) * w, axis=-1, keepdims=True)
        cm = jnp.max(scores, axis=1, keepdims=True)
        m_new = cm if m is None else jnp.maximum(m, cm)
        p = jnp.exp2(scores - m_new)
        pd = jnp.sum(p, axis=1, keepdims=True)
        pacc = jnp.sum(xc * p, axis=1, keepdims=True)
        if m is None:
            d, acc = pd, pacc
        else:
            alpha = jnp.exp2(m - m_new)
            d = d * alpha + pd
            acc = acc * alpha + pacc
        m = m_new
    return (acc / d).reshape(1, bb, h)


def _attn_pool_kernel(x_hbm, w_ref, o_ref, buf, sems, *, bb: int, sc: int,
                      ksplit: int):
    nblocks = x_hbm.shape[0] // bb
    rk = bb // ksplit
    w = w_ref[...][None, :, :]

    def copy(j, k):
        slot = jax.lax.rem(j, _NBUF)
        base = pl.multiple_of(j * bb, bb)
        return pltpu.make_async_copy(
            x_hbm.at[pl.ds(base + k * rk, rk)],
            buf.at[slot].at[pl.ds(k * rk, rk)],
            sems.at[slot, k],
        )

    def start(j):
        for k in range(ksplit):
            copy(j, k).start()

    def wait(j):
        for k in range(ksplit):
            copy(j, k).wait()

    for j in range(_NBUF):
        start(j)

    def body(i, _):
        wait(i)
        slot = jax.lax.rem(i, _NBUF)
        o_ref[pl.ds(i, 1), :, :] = _online_block(buf.at[slot], w, sc)

        @pl.when(i + _NBUF < nblocks)
        def _():
            start(i + _NBUF)

        return ()

    jax.lax.fori_loop(0, nblocks, body, (), unroll=False)


def kernel(encoder_outputs, attention_weights):
    b, s, h = encoder_outputs.shape
    w2 = (attention_weights * jnp.float32(1.4426950408889634)).reshape(1, h)

    bb = 8
    sc = min(s, 128)
    ksplit = 4
    assert b % bb == 0 and s % sc == 0 and bb % ksplit == 0
    nblocks = b // bb

    out = pl.pallas_call(
        functools.partial(_attn_pool_kernel, bb=bb, sc=sc, ksplit=ksplit),
        out_shape=jax.ShapeDtypeStruct((nblocks, bb, h), jnp.float32),
        in_specs=[
            pl.BlockSpec(memory_space=pl.ANY),
            pl.BlockSpec(memory_space=pltpu.VMEM),
        ],
        out_specs=pl.BlockSpec(memory_space=pltpu.VMEM),
        scratch_shapes=[
            pltpu.VMEM((_NBUF, bb, s, h), jnp.float32),
            pltpu.SemaphoreType.DMA((_NBUF, ksplit)),
        ],
        compiler_params=pltpu.CompilerParams(
            vmem_limit_bytes=56 * 1024 * 1024,
        ),
        name="additive_attention_pool",
    )(encoder_outputs, w2)
    return out.reshape(b, h)


# speedup vs baseline: 1.0317x; 1.0317x over previous
import functools

import jax
import jax.numpy as jnp
from jax.experimental import pallas as pl
from jax.experimental.pallas import tpu as pltpu

_NBUF = 3


def _online_block(x_view, w, sc: int):
    bb, s, h = x_view.shape
    nc = s // sc
    m = d = acc = None
    for c in range(nc):
        xc = x_view[:, c * sc:(c + 1) * sc, :]
        scores = jnp.sum(jnp.tanh(xc) * w, axis=-1, keepdims=True)
        cm = jnp.max(scores, axis=1, keepdims=True)
        m_new = cm if m is None else jnp.maximum(m, cm)
        p = jnp.exp2(scores - m_new)
        pd = jnp.sum(p, axis=1, keepdims=True)
        pacc = jnp.sum(xc * p, axis=1, keepdims=True)
        if m is None:
            d, acc = pd, pacc
        else:
            alpha = jnp.exp2(m - m_new)
            d = d * alpha + pd
            acc = acc * alpha + pacc
        m = m_new
    return (acc / d).reshape(1, bb, h)


def _attn_pool_kernel(x_hbm, w_ref, o_ref, buf, sems, *, bb: int, sc: int,
                      ksplit: int):
    nblocks = x_hbm.shape[0] // bb
    rk = bb // ksplit
    w = w_ref[...][None, :, :]

    def copy(j, k):
        slot = jax.lax.rem(j, _NBUF)
        base = pl.multiple_of(j * bb, bb)
        return pltpu.make_async_copy(
            x_hbm.at[pl.ds(base + k * rk, rk)],
            buf.at[slot].at[pl.ds(k * rk, rk)],
            sems.at[slot, k],
        )

    def start(j):
        for k in range(ksplit):
            copy(j, k).start()

    def wait(j):
        for k in range(ksplit):
            copy(j, k).wait()

    for j in range(_NBUF):
        start(j)

    def body(i, _):
        wait(i)
        slot = jax.lax.rem(i, _NBUF)
        o_ref[pl.ds(i, 1), :, :] = _online_block(buf.at[slot], w, sc)

        @pl.when(i + _NBUF < nblocks)
        def _():
            start(i + _NBUF)

        return ()

    jax.lax.fori_loop(0, nblocks, body, (), unroll=False)


def kernel(encoder_outputs, attention_weights):
    b, s, h = encoder_outputs.shape
    w2 = (attention_weights * jnp.float32(1.4426950408889634)).reshape(1, h)

    bb = 8
    sc = min(s, 256)
    ksplit = 4
    assert b % bb == 0 and s % sc == 0 and bb % ksplit == 0
    nblocks = b // bb

    out = pl.pallas_call(
        functools.partial(_attn_pool_kernel, bb=bb, sc=sc, ksplit=ksplit),
        out_shape=jax.ShapeDtypeStruct((nblocks, bb, h), jnp.float32),
        in_specs=[
            pl.BlockSpec(memory_space=pl.ANY),
            pl.BlockSpec(memory_space=pltpu.VMEM),
        ],
        out_specs=pl.BlockSpec(memory_space=pltpu.VMEM),
        scratch_shapes=[
            pltpu.VMEM((_NBUF, bb, s, h), jnp.float32),
            pltpu.SemaphoreType.DMA((_NBUF, ksplit)),
        ],
        compiler_params=pltpu.CompilerParams(
            vmem_limit_bytes=56 * 1024 * 1024,
        ),
        name="additive_attention_pool",
    )(encoder_outputs, w2)
    return out.reshape(b, h)


# speedup vs baseline: 1.0896x; 1.0561x over previous
import functools

import jax
import jax.numpy as jnp
from jax.experimental import pallas as pl
from jax.experimental.pallas import tpu as pltpu

_NBUF = 3


def _online_block(x_view, w, sc: int):
    bb, s, h = x_view.shape
    nc = s // sc
    m = d = acc = None
    for c in range(nc):
        xc = x_view[:, c * sc:(c + 1) * sc, :]
        scores = jnp.sum(jnp.tanh(xc) * w, axis=-1, keepdims=True)
        cm = jnp.max(scores, axis=1, keepdims=True)
        if m is None:
            base = cm
            p = jnp.exp2(scores - base)
            d = jnp.sum(p, axis=1, keepdims=True)
            acc = jnp.sum(xc * p, axis=1, keepdims=True)
        else:
            base = m
            p = jnp.exp2(scores - base)
            pd = jnp.sum(p, axis=1, keepdims=True)
            pacc = jnp.sum(xc * p, axis=1, keepdims=True)
            m_new = jnp.maximum(m, cm)
            alpha = jnp.exp2(base - m_new)
            d = (d + pd) * alpha
            acc = (acc + pacc) * alpha
        m = m_new if c > 0 else cm
    return (acc / d).reshape(1, bb, h)


def _attn_pool_kernel(x_hbm, w_ref, o_ref, buf, sems, *, bb: int, sc: int,
                      ksplit: int):
    nblocks = x_hbm.shape[0] // bb
    rk = bb // ksplit
    w = w_ref[...][None, :, :]

    def copy(j, k):
        slot = jax.lax.rem(j, _NBUF)
        base = pl.multiple_of(j * bb, bb)
        return pltpu.make_async_copy(
            x_hbm.at[pl.ds(base + k * rk, rk)],
            buf.at[slot].at[pl.ds(k * rk, rk)],
            sems.at[slot, k],
        )

    def start(j):
        for k in range(ksplit):
            copy(j, k).start()

    def wait(j):
        for k in range(ksplit):
            copy(j, k).wait()

    for j in range(_NBUF):
        start(j)

    def body(i, _):
        wait(i)
        slot = jax.lax.rem(i, _NBUF)
        o_ref[pl.ds(i, 1), :, :] = _online_block(buf.at[slot], w, sc)

        @pl.when(i + _NBUF < nblocks)
        def _():
            start(i + _NBUF)

        return ()

    jax.lax.fori_loop(0, nblocks, body, (), unroll=False)


def kernel(encoder_outputs, attention_weights):
    b, s, h = encoder_outputs.shape
    w2 = (attention_weights * jnp.float32(1.4426950408889634)).reshape(1, h)

    bb = 8
    sc = min(s, 256)
    ksplit = 4
    assert b % bb == 0 and s % sc == 0 and bb % ksplit == 0
    nblocks = b // bb

    out = pl.pallas_call(
        functools.partial(_attn_pool_kernel, bb=bb, sc=sc, ksplit=ksplit),
        out_shape=jax.ShapeDtypeStruct((nblocks, bb, h), jnp.float32),
        in_specs=[
            pl.BlockSpec(memory_space=pl.ANY),
            pl.BlockSpec(memory_space=pltpu.VMEM),
        ],
        out_specs=pl.BlockSpec(memory_space=pltpu.VMEM),
        scratch_shapes=[
            pltpu.VMEM((_NBUF, bb, s, h), jnp.float32),
            pltpu.SemaphoreType.DMA((_NBUF, ksplit)),
        ],
        compiler_params=pltpu.CompilerParams(
            vmem_limit_bytes=56 * 1024 * 1024,
        ),
        name="additive_attention_pool",
    )(encoder_outputs, w2)
    return out.reshape(b, h)
